# baseline (speedup 1.0000x reference)
import functools

import numpy as np
import jax
import jax.numpy as jnp
from jax import lax
from jax.experimental import pallas as pl
from jax.experimental.pallas import tpu as pltpu

_KS2 = np.int32(0x1BD11BDA)
_MANT_ONE = np.int32(0x3F800000)
_TINY = np.float32(np.finfo(np.float32).tiny)
_ROT_A = (13, 15, 26, 6)
_ROT_B = (17, 29, 16, 24)


def _rotl(v, r):
    return (v << np.int32(r)) | lax.shift_right_logical(v, np.int32(32 - r))


def _rounds(x0, x1, rots):
    for r in rots:
        x0 = x0 + x1
        x1 = _rotl(x1, r)
        x1 = x1 ^ x0
    return x0, x1


def _threefry_bits(j):
    x0 = jnp.zeros_like(j)
    x1 = j
    x0, x1 = _rounds(x0, x1, _ROT_A)
    x1 = x1 + np.int32(_KS2 + 1)
    x0, x1 = _rounds(x0, x1, _ROT_B)
    x0 = x0 + _KS2
    x1 = x1 + np.int32(2)
    x0, x1 = _rounds(x0, x1, _ROT_A)
    x1 = x1 + np.int32(3)
    x0, x1 = _rounds(x0, x1, _ROT_B)
    x1 = x1 + np.int32(_KS2 + 4)
    x0, x1 = _rounds(x0, x1, _ROT_A)
    x0 = x0 + _KS2
    x1 = x1 + np.int32(5)
    return x0 ^ x1


def _gumbel_from_bits(bits):
    mant = lax.shift_right_logical(bits, np.int32(9)) | _MANT_ONE
    u = lax.bitcast_convert_type(mant, jnp.float32) - np.float32(1.0)
    u = u * (np.float32(1.0) - _TINY) + _TINY
    u = jnp.maximum(_TINY, u)
    return -jnp.log(-jnp.log(u))


def _body(x_ref, out_ref, max_ref, idx_ref, *, ncols, width):
    k = pl.program_id(0)
    nsteps = pl.num_programs(0)
    rows = x_ref.shape[0]
    c0 = k * np.int32(width)

    col = lax.broadcasted_iota(jnp.int32, (rows, width), 1) + c0
    row = lax.broadcasted_iota(jnp.int32, (rows, width), 0)
    j = row * np.int32(ncols) + col

    y = _gumbel_from_bits(_threefry_bits(j)) + x_ref[...]
    y = jnp.where(col < np.int32(ncols), y, -jnp.inf)

    loc_max = jnp.max(y, axis=1, keepdims=True)
    loc_idx = jnp.min(
        jnp.where(y == loc_max, col, np.int32(np.iinfo(np.int32).max)),
        axis=1, keepdims=True)

    @pl.when(k == 0)
    def _():
        max_ref[...] = loc_max
        idx_ref[...] = loc_idx

    @pl.when(k > 0)
    def _():
        upd = loc_max > max_ref[...]
        max_ref[...] = jnp.where(upd, loc_max, max_ref[...])
        idx_ref[...] = jnp.where(upd, loc_idx, idx_ref[...])

    @pl.when(k == nsteps - 1)
    def _():
        out_ref[...] = idx_ref[...]


def kernel(x):
    rows, ncols = x.shape
    width = 8192
    nsteps = pl.cdiv(ncols, width)
    out = pl.pallas_call(
        functools.partial(_body, ncols=ncols, width=width),
        grid=(nsteps,),
        in_specs=[pl.BlockSpec((rows, width), lambda k: (0, k))],
        out_specs=pl.BlockSpec((rows, 1), lambda k: (0, 0)),
        out_shape=jax.ShapeDtypeStruct((rows, 1), jnp.int32),
        scratch_shapes=[
            pltpu.VMEM((rows, 1), jnp.float32),
            pltpu.VMEM((rows, 1), jnp.int32),
        ],
    )(x)
    return out.reshape(rows)


# speedup vs baseline: 1.3138x; 1.3138x over previous
import functools

import numpy as np
import jax
import jax.numpy as jnp
from jax import lax
from jax.experimental import pallas as pl
from jax.experimental.pallas import tpu as pltpu

_KS2 = np.int32(0x1BD11BDA)
_MANT_ONE = np.int32(0x3F800000)
_TINY = np.float32(np.finfo(np.float32).tiny)
_BIG = np.int32(np.iinfo(np.int32).max)


def _rotl(v, r):
    return (v << np.int32(r)) | lax.shift_right_logical(v, np.int32(32 - r))


def _threefry_bits(j):
    x0 = j
    x1 = _rotl(j, 13) ^ j
    for r in (15, 26, 6):
        x0 = x0 + x1
        x1 = _rotl(x1, r)
        x1 = x1 ^ x0
    x1 = x1 + np.int32(_KS2 + 1)
    for r in (17, 29, 16, 24):
        x0 = x0 + x1
        x1 = _rotl(x1, r)
        x1 = x1 ^ x0
    x0 = x0 + _KS2
    x1 = x1 + np.int32(2)
    for r in (13, 15, 26, 6):
        x0 = x0 + x1
        x1 = _rotl(x1, r)
        x1 = x1 ^ x0
    x1 = x1 + np.int32(3)
    for r in (17, 29, 16, 24):
        x0 = x0 + x1
        x1 = _rotl(x1, r)
        x1 = x1 ^ x0
    x1 = x1 + np.int32(_KS2 + 4)
    for r in (13, 15, 26, 6):
        x0 = x0 + x1
        x1 = _rotl(x1, r)
        x1 = x1 ^ x0
    x0 = x0 + _KS2
    x1 = x1 + np.int32(5)
    return x0 ^ x1


def _gumbel_from_bits(bits):
    mant = lax.shift_right_logical(bits, np.int32(9)) | _MANT_ONE
    u = lax.bitcast_convert_type(mant, jnp.float32) - np.float32(1.0)
    u = u * (np.float32(1.0) - _TINY) + _TINY
    u = jnp.maximum(_TINY, u)
    return -jnp.log(-jnp.log(u))


def _body(x_ref, out_ref, max_ref, idx_ref, *, ncols, width, sub):
    k = pl.program_id(0)
    nsteps = pl.num_programs(0)
    rows = x_ref.shape[0]
    c0 = k * np.int32(width)

    lane = lax.broadcasted_iota(jnp.int32, (rows, sub), 1)
    row_base = lax.broadcasted_iota(jnp.int32, (rows, sub), 0) * np.int32(ncols)
    j0 = row_base + (lane + c0)
    row_limit = row_base + np.int32(ncols)

    def inner(i, carry):
        ymax, argj = carry
        start = pl.multiple_of(i * sub, sub)
        xs = x_ref[:, pl.ds(start, sub)]
        jv = j0 + i * np.int32(sub)
        y = _gumbel_from_bits(_threefry_bits(jv)) + xs
        upd = (y > ymax) & (jv < row_limit)
        return jnp.where(upd, y, ymax), jnp.where(upd, jv, argj)

    init = (jnp.full((rows, sub), -jnp.inf, jnp.float32),
            jnp.zeros((rows, sub), jnp.int32))
    ymax, argj = lax.fori_loop(0, width // sub, inner, init)

    loc_max = jnp.max(ymax, axis=1, keepdims=True)
    loc_j = jnp.min(jnp.where(ymax == loc_max, argj, _BIG),
                    axis=1, keepdims=True)
    loc_idx = loc_j - (lax.broadcasted_iota(jnp.int32, (rows, 1), 0)
                       * np.int32(ncols))

    @pl.when(k == 0)
    def _():
        max_ref[...] = loc_max
        idx_ref[...] = loc_idx

    @pl.when(k > 0)
    def _():
        upd = loc_max > max_ref[...]
        max_ref[...] = jnp.where(upd, loc_max, max_ref[...])
        idx_ref[...] = jnp.where(upd, loc_idx, idx_ref[...])

    @pl.when(k == nsteps - 1)
    def _():
        out_ref[...] = idx_ref[...]


def kernel(x):
    rows, ncols = x.shape
    width = 8192
    sub = 128
    nsteps = pl.cdiv(ncols, width)
    out = pl.pallas_call(
        functools.partial(_body, ncols=ncols, width=width, sub=sub),
        grid=(nsteps,),
        in_specs=[pl.BlockSpec((rows, width), lambda k: (0, k))],
        out_specs=pl.BlockSpec((rows, 1), lambda k: (0, 0)),
        out_shape=jax.ShapeDtypeStruct((rows, 1), jnp.int32),
        scratch_shapes=[
            pltpu.VMEM((rows, 1), jnp.float32),
            pltpu.VMEM((rows, 1), jnp.int32),
        ],
    )(x)
    return out.reshape(rows)


# speedup vs baseline: 1.3616x; 1.0364x over previous
import functools

import numpy as np
import jax
import jax.numpy as jnp
from jax import lax
from jax.experimental import pallas as pl
from jax.experimental.pallas import tpu as pltpu

_KS2 = np.int32(0x1BD11BDA)
_MANT_ONE = np.int32(0x3F800000)
_TINY = np.float32(np.finfo(np.float32).tiny)
_BIG = np.int32(np.iinfo(np.int32).max)


def _rotl(v, r):
    return (v << np.int32(r)) | lax.shift_right_logical(v, np.int32(32 - r))


def _threefry_bits(j):
    x0 = j
    x1 = _rotl(j, 13) ^ j
    for r in (15, 26, 6):
        x0 = x0 + x1
        x1 = _rotl(x1, r)
        x1 = x1 ^ x0
    x1 = x1 + np.int32(_KS2 + 1)
    for r in (17, 29, 16, 24):
        x0 = x0 + x1
        x1 = _rotl(x1, r)
        x1 = x1 ^ x0
    x0 = x0 + _KS2
    x1 = x1 + np.int32(2)
    for r in (13, 15, 26, 6):
        x0 = x0 + x1
        x1 = _rotl(x1, r)
        x1 = x1 ^ x0
    x1 = x1 + np.int32(3)
    for r in (17, 29, 16, 24):
        x0 = x0 + x1
        x1 = _rotl(x1, r)
        x1 = x1 ^ x0
    x1 = x1 + np.int32(_KS2 + 4)
    for r in (13, 15, 26, 6):
        x0 = x0 + x1
        x1 = _rotl(x1, r)
        x1 = x1 ^ x0
    x0 = x0 + _KS2
    x1 = x1 + np.int32(5)
    return x0 ^ x1


def _gumbel_from_bits(bits):
    mant = lax.shift_right_logical(bits, np.int32(9)) | _MANT_ONE
    u = lax.bitcast_convert_type(mant, jnp.float32) - np.float32(1.0)
    u = u * (np.float32(1.0) - _TINY) + _TINY
    u = jnp.maximum(_TINY, u)
    return -jnp.log(-jnp.log(u))


def _body(x_ref, out_ref, max_ref, idx_ref, *, ncols, width, sub):
    k = pl.program_id(0)
    nsteps = pl.num_programs(0)
    rows = x_ref.shape[0]
    c0 = k * np.int32(width)

    lane = lax.broadcasted_iota(jnp.int32, (rows, sub), 1)
    row_base = lax.broadcasted_iota(jnp.int32, (rows, sub), 0) * np.int32(ncols)
    j0 = row_base + (lane + c0)
    row_limit = row_base + np.int32(ncols)

    def inner(i, carry):
        ymax, argj = carry
        start = pl.multiple_of(i * sub, sub)
        xs = x_ref[:, pl.ds(start, sub)]
        jv = j0 + i * np.int32(sub)
        y = _gumbel_from_bits(_threefry_bits(jv)) + xs
        upd = (y > ymax) & (jv < row_limit)
        return jnp.where(upd, y, ymax), jnp.where(upd, jv, argj)

    init = (jnp.full((rows, sub), -jnp.inf, jnp.float32),
            jnp.zeros((rows, sub), jnp.int32))
    ymax, argj = lax.fori_loop(0, width // sub, inner, init)

    loc_max = jnp.max(ymax, axis=1, keepdims=True)
    loc_j = jnp.min(jnp.where(ymax == loc_max, argj, _BIG),
                    axis=1, keepdims=True)
    loc_idx = loc_j - (lax.broadcasted_iota(jnp.int32, (rows, 1), 0)
                       * np.int32(ncols))

    @pl.when(k == 0)
    def _():
        max_ref[...] = loc_max
        idx_ref[...] = loc_idx

    @pl.when(k > 0)
    def _():
        upd = loc_max > max_ref[...]
        max_ref[...] = jnp.where(upd, loc_max, max_ref[...])
        idx_ref[...] = jnp.where(upd, loc_idx, idx_ref[...])

    @pl.when(k == nsteps - 1)
    def _():
        out_ref[...] = idx_ref[...]


def kernel(x):
    rows, ncols = x.shape
    width = 8192
    sub = 256
    nsteps = pl.cdiv(ncols, width)
    out = pl.pallas_call(
        functools.partial(_body, ncols=ncols, width=width, sub=sub),
        grid=(nsteps,),
        in_specs=[pl.BlockSpec((rows, width), lambda k: (0, k))],
        out_specs=pl.BlockSpec((rows, 1), lambda k: (0, 0)),
        out_shape=jax.ShapeDtypeStruct((rows, 1), jnp.int32),
        scratch_shapes=[
            pltpu.VMEM((rows, 1), jnp.float32),
            pltpu.VMEM((rows, 1), jnp.int32),
        ],
    )(x)
    return out.reshape(rows)
